# baseline (speedup 1.0000x reference)
import functools

import jax
import jax.numpy as jnp
from jax import lax
from jax.experimental import pallas as pl
from jax.experimental.pallas import tpu as pltpu
from jax.experimental.pallas import tpu_sc as plsc

BATCH = 4096
HIST = 50
E_DIMS = 64
CHUNK_B = 2
CHUNK_I = CHUNK_B * HIST
CHUNK_IP = 104
NBUF = 4

_info = plsc.get_sparse_core_info()
_NC, _NS = _info.num_cores, _info.num_subcores
_NW = _NC * _NS
_ROWS_W = BATCH // _NW
_CHUNKS_W = _ROWS_W // CHUNK_B

_mesh = plsc.VectorSubcoreMesh(core_axis_name="c", subcore_axis_name="s")


@functools.partial(
    pl.kernel,
    mesh=_mesh,
    out_type=jax.ShapeDtypeStruct((BATCH, HIST, E_DIMS), jnp.float32),
    scratch_types=[
        pltpu.VMEM((_CHUNKS_W, CHUNK_IP), jnp.int32),
        [pltpu.VMEM((CHUNK_IP, E_DIMS), jnp.float32) for _ in range(NBUF)],
        [pltpu.SemaphoreType.DMA for _ in range(NBUF)],
        [pltpu.SemaphoreType.DMA for _ in range(NBUF)],
    ],
    compiler_params=pltpu.CompilerParams(use_tc_tiling_on_sc=False),
)
def _gather_kernel(tab_hbm, xp_hbm, out_hbm, idx_v, bufs, gsems, wsems):
    wid = lax.axis_index("s") * _NC + lax.axis_index("c")
    b0 = wid * _ROWS_W
    pltpu.sync_copy(xp_hbm.at[wid], idx_v)

    def start_gather(c, k):
        pltpu.make_async_copy(tab_hbm.at[idx_v.at[c]], bufs[k], gsems[k]).start()

    def wait_gather(c, k):
        pltpu.make_async_copy(tab_hbm.at[idx_v.at[c]], bufs[k], gsems[k]).wait()

    def start_write(c, k):
        b = b0 + CHUNK_B * c
        pltpu.make_async_copy(
            bufs[k].at[pl.ds(0, HIST), :], out_hbm.at[b], wsems[k]
        ).start()
        pltpu.make_async_copy(
            bufs[k].at[pl.ds(HIST, HIST), :], out_hbm.at[b + 1], wsems[k]
        ).start()

    def wait_write(c, k):
        b = b0 + CHUNK_B * c
        pltpu.make_async_copy(
            bufs[k].at[pl.ds(0, HIST), :], out_hbm.at[b], wsems[k]
        ).wait()
        pltpu.make_async_copy(
            bufs[k].at[pl.ds(HIST, HIST), :], out_hbm.at[b + 1], wsems[k]
        ).wait()

    start_gather(0, 0)
    start_gather(1, 1)

    def body(i, carry):
        for k in range(NBUF):
            c = NBUF * i + k
            k2 = (k + 2) % NBUF
            wait_gather(c, k)
            start_write(c, k)

            @pl.when(c + 2 < _CHUNKS_W)
            def _():
                @pl.when(c >= 2)
                def _():
                    wait_write(c - 2, k2)

                start_gather(c + 2, k2)

        return carry

    lax.fori_loop(0, _CHUNKS_W // NBUF, body, 0)
    wait_write(_CHUNKS_W - 2, (_CHUNKS_W - 2) % NBUF)
    wait_write(_CHUNKS_W - 1, (_CHUNKS_W - 1) % NBUF)


def kernel(x, E_object_index):
    x2 = x.astype(jnp.int32).reshape(BATCH // CHUNK_B, CHUNK_I)
    xp = jnp.concatenate([x2, x2[:, CHUNK_I - (CHUNK_IP - CHUNK_I):]], axis=1)
    xp = xp.reshape(_NW, _CHUNKS_W, CHUNK_IP)
    xp = jnp.maximum(xp, 0)
    return _gather_kernel(E_object_index, xp)


# speedup vs baseline: 1.1071x; 1.1071x over previous
import functools

import jax
import jax.numpy as jnp
from jax import lax
from jax.experimental import pallas as pl
from jax.experimental.pallas import tpu as pltpu
from jax.experimental.pallas import tpu_sc as plsc

BATCH = 4096
HIST = 50
E_DIMS = 64
PAD_DIMS = 128
CHUNK_B = 2
CHUNK_I = CHUNK_B * HIST
CHUNK_IP = 104
NBUF = 3
LANES = 16

_info = plsc.get_sparse_core_info()
_NC, _NS = _info.num_cores, _info.num_subcores
_NW = _NC * _NS
_ROWS_W = BATCH // _NW
_CHUNKS_W = _ROWS_W // CHUNK_B

_mesh = plsc.VectorSubcoreMesh(core_axis_name="c", subcore_axis_name="s")


@functools.partial(
    pl.kernel,
    mesh=_mesh,
    out_type=jax.ShapeDtypeStruct((BATCH, HIST, E_DIMS), jnp.float32),
    scratch_types=[
        pltpu.VMEM((_CHUNKS_W, CHUNK_IP), jnp.int32),
        [pltpu.VMEM((CHUNK_IP, PAD_DIMS), jnp.float32) for _ in range(NBUF)],
        [
            [pltpu.VMEM((HIST, E_DIMS), jnp.float32) for _ in range(CHUNK_B)]
            for _ in range(NBUF)
        ],
        [pltpu.SemaphoreType.DMA for _ in range(NBUF)],
        [pltpu.SemaphoreType.DMA for _ in range(NBUF)],
    ],
)
def _gather_kernel(tab_hbm, xp_hbm, out_hbm, idx_v, bufa, bufb, gsems, wsems):
    wid = lax.axis_index("s") * _NC + lax.axis_index("c")
    b0 = wid * _ROWS_W
    pltpu.sync_copy(xp_hbm.at[wid], idx_v)

    def start_gather(c, k):
        pltpu.make_async_copy(tab_hbm.at[idx_v.at[c]], bufa[k], gsems[k]).start()

    def wait_gather(c, k):
        pltpu.make_async_copy(tab_hbm.at[idx_v.at[c]], bufa[k], gsems[k]).wait()

    def extract(k):
        for rb in range(CHUNK_B):
            for r in range(HIST):
                for v in range(E_DIMS // LANES):
                    bufb[k][rb][r, pl.ds(v * LANES, LANES)] = bufa[k][
                        rb * HIST + r, pl.ds(v * LANES, LANES)
                    ]

    def start_write(c, k):
        b = b0 + CHUNK_B * c
        for rb in range(CHUNK_B):
            pltpu.make_async_copy(bufb[k][rb], out_hbm.at[b + rb], wsems[k]).start()

    def wait_write(c, k):
        b = b0 + CHUNK_B * c
        for rb in range(CHUNK_B):
            pltpu.make_async_copy(bufb[k][rb], out_hbm.at[b + rb], wsems[k]).wait()

    for k in range(NBUF):
        start_gather(k, k)

    def body(i, carry):
        for k in range(NBUF):
            c = NBUF * i + k
            wait_gather(c, k)

            @pl.when(c >= NBUF)
            def _():
                wait_write(c - NBUF, k)

            extract(k)
            start_write(c, k)

            @pl.when(c + NBUF < _CHUNKS_W)
            def _():
                start_gather(c + NBUF, k)

        return carry

    lax.fori_loop(0, _CHUNKS_W // NBUF, body, 0)
    c_last = (_CHUNKS_W // NBUF) * NBUF
    k_last = c_last % NBUF
    wait_gather(c_last, k_last)
    wait_write(c_last - NBUF, k_last)
    extract(k_last)
    start_write(c_last, k_last)
    for c in range(c_last + 1, c_last + NBUF + 1):
        wait_write(c - NBUF, c % NBUF)


def kernel(x, E_object_index):
    tab = jnp.pad(E_object_index, ((0, 0), (0, PAD_DIMS - E_DIMS)))
    x2 = x.astype(jnp.int32).reshape(BATCH // CHUNK_B, CHUNK_I)
    xp = jnp.concatenate([x2, x2[:, CHUNK_I - (CHUNK_IP - CHUNK_I):]], axis=1)
    xp = xp.reshape(_NW, _CHUNKS_W, CHUNK_IP)
    return _gather_kernel(tab, xp)


# speedup vs baseline: 1.1071x; 1.0000x over previous
import functools

import jax
import jax.numpy as jnp
from jax import lax
from jax.experimental import pallas as pl
from jax.experimental.pallas import tpu as pltpu
from jax.experimental.pallas import tpu_sc as plsc

BATCH = 4096
HIST = 50
E_DIMS = 64
PAD_DIMS = 128
CHUNK_B = 2
CHUNK_I = CHUNK_B * HIST
CHUNK_IP = 104
CHUNK_ROW = 128
NBUF = 3
LANES = 16

_info = plsc.get_sparse_core_info()
_NC, _NS = _info.num_cores, _info.num_subcores
_NW = _NC * _NS
_ROWS_W = BATCH // _NW
_CHUNKS_W = _ROWS_W // CHUNK_B

_mesh = plsc.VectorSubcoreMesh(core_axis_name="c", subcore_axis_name="s")


@functools.partial(
    pl.kernel,
    mesh=_mesh,
    out_type=jax.ShapeDtypeStruct((BATCH, HIST, E_DIMS), jnp.float32),
    scratch_types=[
        pltpu.VMEM((_CHUNKS_W, CHUNK_ROW), jnp.int32),
        [pltpu.VMEM((CHUNK_IP, PAD_DIMS), jnp.float32) for _ in range(NBUF)],
        [
            [pltpu.VMEM((HIST, E_DIMS), jnp.float32) for _ in range(CHUNK_B)]
            for _ in range(NBUF)
        ],
        [pltpu.SemaphoreType.DMA for _ in range(NBUF)],
        [pltpu.SemaphoreType.DMA for _ in range(NBUF)],
    ],
)
def _gather_kernel(tab_hbm, xp_hbm, out_hbm, idx_v, bufa, bufb, gsems, wsems):
    wid = lax.axis_index("s") * _NC + lax.axis_index("c")
    b0 = wid * _ROWS_W
    pltpu.sync_copy(xp_hbm.at[wid], idx_v)

    def start_gather(c, k):
        pltpu.make_async_copy(
            tab_hbm.at[idx_v.at[c, pl.ds(0, CHUNK_IP)]], bufa[k], gsems[k]
        ).start()

    def wait_gather(c, k):
        pltpu.make_async_copy(
            tab_hbm.at[idx_v.at[c, pl.ds(0, CHUNK_IP)]], bufa[k], gsems[k]
        ).wait()

    def extract(k):
        for rb in range(CHUNK_B):
            for r in range(HIST):
                for v in range(E_DIMS // LANES):
                    bufb[k][rb][r, pl.ds(v * LANES, LANES)] = bufa[k][
                        rb * HIST + r, pl.ds(v * LANES, LANES)
                    ]

    def start_write(c, k):
        b = b0 + CHUNK_B * c
        for rb in range(CHUNK_B):
            pltpu.make_async_copy(bufb[k][rb], out_hbm.at[b + rb], wsems[k]).start()

    def wait_write(c, k):
        b = b0 + CHUNK_B * c
        for rb in range(CHUNK_B):
            pltpu.make_async_copy(bufb[k][rb], out_hbm.at[b + rb], wsems[k]).wait()

    for k in range(NBUF):
        start_gather(k, k)

    def body(i, carry):
        for k in range(NBUF):
            c = NBUF * i + k
            wait_gather(c, k)

            @pl.when(c >= NBUF)
            def _():
                wait_write(c - NBUF, k)

            extract(k)
            start_write(c, k)

            @pl.when(c + NBUF < _CHUNKS_W)
            def _():
                start_gather(c + NBUF, k)

        return carry

    lax.fori_loop(0, _CHUNKS_W // NBUF, body, 0)
    c_last = (_CHUNKS_W // NBUF) * NBUF
    k_last = c_last % NBUF
    wait_gather(c_last, k_last)
    wait_write(c_last - NBUF, k_last)
    extract(k_last)
    start_write(c_last, k_last)
    for c in range(c_last + 1, c_last + NBUF + 1):
        wait_write(c - NBUF, c % NBUF)


def kernel(x, E_object_index):
    tab = jnp.pad(E_object_index, ((0, 0), (0, PAD_DIMS - E_DIMS)))
    x2 = x.astype(jnp.int32).reshape(BATCH // CHUNK_B, CHUNK_I)
    xp = jnp.concatenate(
        [
            x2,
            x2[:, CHUNK_I - (CHUNK_IP - CHUNK_I):],
            jnp.zeros((BATCH // CHUNK_B, CHUNK_ROW - CHUNK_IP), jnp.int32),
        ],
        axis=1,
    )
    xp = xp.reshape(_NW, _CHUNKS_W, CHUNK_ROW)
    return _gather_kernel(tab, xp)


# speedup vs baseline: 1.1077x; 1.0005x over previous
import functools

import jax
import jax.numpy as jnp
from jax import lax
from jax.experimental import pallas as pl
from jax.experimental.pallas import tpu as pltpu
from jax.experimental.pallas import tpu_sc as plsc

BATCH = 4096
HIST = 50
E_DIMS = 64
PAD_DIMS = 128
CHUNK_B = 2
CHUNK_I = CHUNK_B * HIST
CHUNK_IP = 104
CHUNK_ROW = 128
NBUF = 4
LANES = 16

_info = plsc.get_sparse_core_info()
_NC, _NS = _info.num_cores, _info.num_subcores
_NW = _NC * _NS
_ROWS_W = BATCH // _NW
_CHUNKS_W = _ROWS_W // CHUNK_B

_mesh = plsc.VectorSubcoreMesh(core_axis_name="c", subcore_axis_name="s")


@functools.partial(
    pl.kernel,
    mesh=_mesh,
    out_type=jax.ShapeDtypeStruct((BATCH, HIST, E_DIMS), jnp.float32),
    scratch_types=[
        pltpu.VMEM((_CHUNKS_W, CHUNK_ROW), jnp.int32),
        [pltpu.VMEM((CHUNK_IP, PAD_DIMS), jnp.float32) for _ in range(NBUF)],
        [
            [pltpu.VMEM((HIST, E_DIMS), jnp.float32) for _ in range(CHUNK_B)]
            for _ in range(NBUF)
        ],
        [pltpu.SemaphoreType.DMA for _ in range(NBUF)],
        [pltpu.SemaphoreType.DMA for _ in range(NBUF)],
    ],
)
def _gather_kernel(tab_hbm, xp_hbm, out_hbm, idx_v, bufa, bufb, gsems, wsems):
    wid = lax.axis_index("s") * _NC + lax.axis_index("c")
    b0 = wid * _ROWS_W
    pltpu.sync_copy(xp_hbm.at[wid], idx_v)

    def start_gather(c, k):
        pltpu.make_async_copy(
            tab_hbm.at[idx_v.at[c, pl.ds(0, CHUNK_IP)]], bufa[k], gsems[k]
        ).start()

    def wait_gather(c, k):
        pltpu.make_async_copy(
            tab_hbm.at[idx_v.at[c, pl.ds(0, CHUNK_IP)]], bufa[k], gsems[k]
        ).wait()

    def extract(k):
        for rb in range(CHUNK_B):
            for r in range(HIST):
                for v in range(E_DIMS // LANES):
                    bufb[k][rb][r, pl.ds(v * LANES, LANES)] = bufa[k][
                        rb * HIST + r, pl.ds(v * LANES, LANES)
                    ]

    def start_write(c, k):
        b = b0 + CHUNK_B * c
        for rb in range(CHUNK_B):
            pltpu.make_async_copy(bufb[k][rb], out_hbm.at[b + rb], wsems[k]).start()

    def wait_write(c, k):
        b = b0 + CHUNK_B * c
        for rb in range(CHUNK_B):
            pltpu.make_async_copy(bufb[k][rb], out_hbm.at[b + rb], wsems[k]).wait()

    for k in range(NBUF):
        start_gather(k, k)

    def body(i, carry):
        for k in range(NBUF):
            c = NBUF * i + k
            wait_gather(c, k)

            @pl.when(c >= NBUF)
            def _():
                wait_write(c - NBUF, k)

            extract(k)
            start_write(c, k)

            @pl.when(c + NBUF < _CHUNKS_W)
            def _():
                start_gather(c + NBUF, k)

        return carry

    lax.fori_loop(0, _CHUNKS_W // NBUF, body, 0)
    for c in range(_CHUNKS_W - NBUF, _CHUNKS_W):
        wait_write(c, c % NBUF)


def kernel(x, E_object_index):
    tab = jnp.pad(E_object_index, ((0, 0), (0, PAD_DIMS - E_DIMS)))
    x2 = x.astype(jnp.int32).reshape(BATCH // CHUNK_B, CHUNK_I)
    xp = jnp.concatenate(
        [
            x2,
            x2[:, CHUNK_I - (CHUNK_IP - CHUNK_I):],
            jnp.zeros((BATCH // CHUNK_B, CHUNK_ROW - CHUNK_IP), jnp.int32),
        ],
        axis=1,
    )
    xp = xp.reshape(_NW, _CHUNKS_W, CHUNK_ROW)
    return _gather_kernel(tab, xp)
